# baseline (speedup 1.0000x reference)
import functools

import jax
import jax.numpy as jnp
from jax.experimental import pallas as pl

N = 10000
C = 16
F = 256
H = 256
O = 256
GRID = 3
ORDER = 2
NB = GRID + ORDER
NK = GRID + 2 * ORDER + 1


def _dot_t(a, w):
    return jax.lax.dot_general(
        a, w, (((1,), (1,)), ((), ())), preferred_element_type=jnp.float32
    )


def _fused_kernel(cma_ref, cmb_ref, aw_ref, x_ref, wn_ref, bn_ref, wc_ref,
                  bc_ref, wu_ref, bu_ref, wb_ref, wsp_ref, grid_ref, out_ref):
    aw3 = aw_ref[...][:, :, None]
    h8 = cma_ref[...] * aw3[:, 0:8] + cmb_ref[...] * aw3[:, 8:16]
    cr = jnp.sum(h8, axis=1) * (1.0 / C)
    aw = aw_ref[...] * (1.0 / C)
    am = jnp.sum(aw, axis=1, keepdims=True)

    h = _dot_t(x_ref[...], wn_ref[...]) + bn_ref[...][None, :]
    h = h + _dot_t(cr, wc_ref[...]) + am * bc_ref[...][None, :]
    u = _dot_t(h, wu_ref[...]) + bu_ref[...][None, :]

    base = _dot_t(u * jax.nn.sigmoid(u), wb_ref[...])

    t0 = grid_ref[0:1, 0:1]
    h_inv = 1.0 / (grid_ref[0:1, 1:2] - t0)
    s = (u - t0) * h_inv
    p = []
    for j in range(NK):
        r = jnp.maximum(s - float(j), 0.0)
        p.append(r * r)
    acc = base
    for j in range(NB):
        bj = 0.5 * ((p[j] - p[j + 3]) - 3.0 * (p[j + 1] - p[j + 2]))
        acc = acc + _dot_t(bj, wsp_ref[j])
    out_ref[...] = acc


@functools.partial(jax.jit, static_argnames=())
def kernel(x, context_map, attention_weights_map, W_node, b_node, W_ctx,
           b_ctx, W_upd, b_upd, kan_base_w, kan_spline_w, kan_grid):
    Tn = 1000
    grid = (N // Tn,)
    wsp = jnp.transpose(kan_spline_w, (2, 0, 1))

    full = lambda *s: pl.BlockSpec(s, lambda i: (0,) * len(s))
    return pl.pallas_call(
        _fused_kernel,
        grid=grid,
        in_specs=[
            pl.BlockSpec((Tn, C // 2, F), lambda i: (i, 0, 0)),
            pl.BlockSpec((Tn, C // 2, F), lambda i: (i, 1, 0)),
            pl.BlockSpec((Tn, C), lambda i: (i, 0)),
            pl.BlockSpec((Tn, F), lambda i: (i, 0)),
            full(H, F), full(H), full(H, F), full(H),
            full(O, H), full(O), full(O, O), full(NB, O, O),
            full(O, NK),
        ],
        out_specs=pl.BlockSpec((Tn, O), lambda i: (i, 0)),
        out_shape=jax.ShapeDtypeStruct((N, O), jnp.float32),
    )(context_map, context_map, attention_weights_map, x, W_node, b_node,
      W_ctx, b_ctx, W_upd, b_upd, kan_base_w, wsp, kan_grid)


# speedup vs baseline: 1.0281x; 1.0281x over previous
import functools

import jax
import jax.numpy as jnp
from jax.experimental import pallas as pl

N = 10000
C = 16
F = 256
H = 256
O = 256
GRID = 3
ORDER = 2
NB = GRID + ORDER
NK = GRID + 2 * ORDER + 1


def _dot_t(a, w):
    return jax.lax.dot_general(
        a, w, (((1,), (1,)), ((), ())), preferred_element_type=jnp.float32
    )


def _fused_kernel(cm_ref, aw_ref, x_ref, wn_ref, bn_ref, wc_ref,
                  bc_ref, wu_ref, bu_ref, wb_ref, wsp_ref, grid_ref, out_ref):
    aw3 = aw_ref[...][:, :, None]
    w3 = cm_ref[...] * aw3
    h8 = w3[:, 0:8, :] + w3[:, 8:16, :]
    cr = jnp.sum(h8, axis=1) * (1.0 / C)
    aw = aw_ref[...] * (1.0 / C)
    am = jnp.sum(aw, axis=1, keepdims=True)

    h = _dot_t(x_ref[...], wn_ref[...]) + bn_ref[...][None, :]
    h = h + _dot_t(cr, wc_ref[...]) + am * bc_ref[...][None, :]
    u = _dot_t(h, wu_ref[...]) + bu_ref[...][None, :]

    base = _dot_t(u * jax.nn.sigmoid(u), wb_ref[...])

    t0 = grid_ref[0:1, 0:1]
    h_inv = 1.0 / (grid_ref[0:1, 1:2] - t0)
    s = (u - t0) * h_inv
    p = []
    for j in range(NK):
        r = jnp.maximum(s - float(j), 0.0)
        p.append(r * r)
    acc = base
    for j in range(NB):
        bj = 0.5 * ((p[j] - p[j + 3]) - 3.0 * (p[j + 1] - p[j + 2]))
        acc = acc + _dot_t(bj, wsp_ref[j])
    out_ref[...] = acc


@functools.partial(jax.jit, static_argnames=())
def kernel(x, context_map, attention_weights_map, W_node, b_node, W_ctx,
           b_ctx, W_upd, b_upd, kan_base_w, kan_spline_w, kan_grid):
    Tn = 1000
    grid = (N // Tn,)
    wsp = jnp.transpose(kan_spline_w, (2, 0, 1))

    full = lambda *s: pl.BlockSpec(s, lambda i: (0,) * len(s))
    return pl.pallas_call(
        _fused_kernel,
        grid=grid,
        in_specs=[
            pl.BlockSpec((Tn, C, F), lambda i: (i, 0, 0)),
            pl.BlockSpec((Tn, C), lambda i: (i, 0)),
            pl.BlockSpec((Tn, F), lambda i: (i, 0)),
            full(H, F), full(H), full(H, F), full(H),
            full(O, H), full(O), full(O, O), full(NB, O, O),
            full(O, NK),
        ],
        out_specs=pl.BlockSpec((Tn, O), lambda i: (i, 0)),
        out_shape=jax.ShapeDtypeStruct((N, O), jnp.float32),
    )(context_map, attention_weights_map, x, W_node, b_node, W_ctx,
      b_ctx, W_upd, b_upd, kan_base_w, wsp, kan_grid)
